# baseline (speedup 1.0000x reference)
import jax
import jax.numpy as jnp
from jax.experimental import pallas as pl
from jax.experimental.pallas import tpu as pltpu

_R = 128
_N = 100000
_C = 12544
_G = 8


def _argmax_kernel(x_ref, out_ref, vmax_ref, vidx_ref):
    j = pl.program_id(0)
    x = x_ref[...]
    cols = j * _C + jax.lax.broadcasted_iota(jnp.int32, (_R, _C), 1)
    x = jnp.where(cols < _N, x, -1.0)
    lmax = jnp.max(x, axis=1, keepdims=True)
    lidx = jnp.min(jnp.where(x == lmax, cols, _N), axis=1, keepdims=True)

    @pl.when(j == 0)
    def _init():
        vmax_ref[...] = lmax
        vidx_ref[...] = lidx

    @pl.when(j > 0)
    def _acc():
        better = lmax > vmax_ref[...]
        vmax_ref[...] = jnp.where(better, lmax, vmax_ref[...])
        vidx_ref[...] = jnp.where(better, lidx, vidx_ref[...])

    @pl.when(j == _G - 1)
    def _fin():
        out_ref[...] = vidx_ref[...]


def kernel(probs):
    out = pl.pallas_call(
        _argmax_kernel,
        grid=(_G,),
        in_specs=[pl.BlockSpec((_R, _C), lambda j: (0, j))],
        out_specs=pl.BlockSpec((_R, 1), lambda j: (0, 0)),
        out_shape=jax.ShapeDtypeStruct((_R, 1), jnp.int32),
        scratch_shapes=[
            pltpu.VMEM((_R, 1), jnp.float32),
            pltpu.VMEM((_R, 1), jnp.int32),
        ],
    )(probs)
    return out[:, 0]
